# baseline (speedup 1.0000x reference)
import functools

import jax
import jax.numpy as jnp
from jax import lax
from jax.experimental import pallas as pl
from jax.experimental.pallas import tpu as pltpu
from jax.experimental.pallas import tpu_sc as plsc

N_NODES = 10000
N_EDGES = 320000
D = 128
L = 16
NC = 2
NS = 16
NW = NC * NS
CHUNK = 128
NCHUNK = N_EDGES // CHUNK
TRIPS = (NCHUNK + NW - 1) // NW
ROWS_LO = 624
ROWS_HI = N_NODES - 15 * ROWS_LO
ZROWS = 16


def _sc_partials(x, src, dst, edge_weight):
    mesh = plsc.VectorSubcoreMesh(
        core_axis_name="c", subcore_axis_name="s", num_cores=NC, num_subcores=NS
    )

    @functools.partial(
        pl.kernel,
        out_type=jax.ShapeDtypeStruct((NC, N_NODES, D), jnp.float32),
        mesh=mesh,
        scratch_types=[
            pltpu.VMEM_SHARED((N_NODES, D), jnp.float32),
            pltpu.VMEM((3, CHUNK), jnp.int32),
            pltpu.VMEM((3, CHUNK), jnp.int32),
            pltpu.VMEM((3, CHUNK), jnp.float32),
            pltpu.VMEM((2, CHUNK, D), jnp.float32),
            pltpu.VMEM((ZROWS, D), jnp.float32),
            pltpu.SemaphoreType.DMA,
            pltpu.SemaphoreType.DMA,
            pltpu.SemaphoreType.DMA,
        ],
    )
    def k(x_hbm, src_hbm, dst_hbm, w_hbm, parts_hbm, acc, idx_s, idx_d, wbuf,
          rows, zbuf, sem_g, sem_i, sem_sc):
        cid = lax.axis_index("c")
        sid = lax.axis_index("s")
        wid = sid * NC + cid
        base_row = sid * ROWS_LO

        zeros = jnp.zeros((L,), jnp.float32)
        for r in range(ZROWS):
            for j in range(D // L):
                zbuf[r, pl.ds(j * L, L)] = zeros

        def zcopy(kk, _):
            pltpu.sync_copy(zbuf, acc.at[pl.ds(base_row + kk * ZROWS, ZROWS)])
            return 0

        n_owned = jnp.where(sid == NS - 1, ROWS_HI, ROWS_LO)
        lax.fori_loop(0, n_owned // ZROWS, zcopy, 0)
        plsc.subcore_barrier()

        def issue_idx(trip, slot):
            base = (trip * NW + wid) * CHUNK
            pltpu.async_copy(src_hbm.at[pl.ds(base, CHUNK)], idx_s.at[slot], sem_i)
            pltpu.async_copy(dst_hbm.at[pl.ds(base, CHUNK)], idx_d.at[slot], sem_i)
            pltpu.async_copy(w_hbm.at[pl.ds(base, CHUNK)], wbuf.at[slot], sem_i)

        def wait_idx(slot):
            pltpu.make_async_copy(src_hbm.at[pl.ds(0, CHUNK)], idx_s.at[slot], sem_i).wait()
            pltpu.make_async_copy(dst_hbm.at[pl.ds(0, CHUNK)], idx_d.at[slot], sem_i).wait()
            pltpu.make_async_copy(w_hbm.at[pl.ds(0, CHUNK)], wbuf.at[slot], sem_i).wait()

        issue_idx(0, 0)
        wait_idx(0)
        pltpu.async_copy(x_hbm.at[idx_s.at[0]], rows.at[0], sem_g)

        def chunk_body(i, _):
            b = lax.rem(i, 2)
            ib = lax.rem(i, 3)
            ib1 = lax.rem(i + 1, 3)
            nxt_valid = (i + 1) * NW + wid < NCHUNK

            pltpu.make_async_copy(x_hbm.at[idx_s.at[ib]], rows.at[b], sem_g).wait()

            @pl.when(nxt_valid)
            def _():
                issue_idx(i + 1, ib1)

            def scale(g, _):
                wg = wbuf[ib, pl.ds(g * L, L)]
                for ee in range(L):
                    e = g * L + ee
                    ws = wg[ee]
                    for j in range(D // L):
                        sl = pl.ds(j * L, L)
                        rows[b, e, sl] = rows[b, e, sl] * ws
                return 0

            lax.fori_loop(0, CHUNK // L, scale, 0)

            @pl.when(i > 0)
            def _():
                pltpu.make_async_copy(
                    rows.at[1 - b], acc.at[idx_d.at[ib]], sem_sc
                ).wait()

            @pl.when(nxt_valid)
            def _():
                wait_idx(ib1)
                pltpu.async_copy(x_hbm.at[idx_s.at[ib1]], rows.at[1 - b], sem_g)

            pltpu.async_copy(rows.at[b], acc.at[idx_d.at[ib]], sem_sc, add=True)
            return 0

        my_trips = jnp.where(wid < NCHUNK - (TRIPS - 1) * NW, TRIPS, TRIPS - 1)
        lax.fori_loop(0, my_trips, chunk_body, 0)
        pltpu.make_async_copy(rows.at[0], acc.at[idx_d.at[0]], sem_sc).wait()
        plsc.subcore_barrier()

        @pl.when(sid < NS - 1)
        def _():
            pltpu.sync_copy(
                acc.at[pl.ds(base_row, ROWS_LO)],
                parts_hbm.at[cid, pl.ds(base_row, ROWS_LO)],
            )

        @pl.when(sid == NS - 1)
        def _():
            pltpu.sync_copy(
                acc.at[pl.ds(15 * ROWS_LO, ROWS_HI)],
                parts_hbm.at[cid, pl.ds(15 * ROWS_LO, ROWS_HI)],
            )

    return k(x, src, dst, edge_weight)


def _combine(p0, p1):
    def body(a_ref, b_ref, o_ref):
        o_ref[...] = jnp.clip(a_ref[...] + b_ref[...], -20.0, 20.0)

    blk = 1000
    spec = pl.BlockSpec((blk, D), lambda i: (i, 0))
    return pl.pallas_call(
        body,
        grid=(N_NODES // blk,),
        in_specs=[spec, spec],
        out_specs=spec,
        out_shape=jax.ShapeDtypeStruct((N_NODES, D), jnp.float32),
    )(p0, p1)


def kernel(t, x, edge_index, edge_weight):
    parts = _sc_partials(x, edge_index[1], edge_index[0], edge_weight)
    return _combine(parts[0], parts[1])


# speedup vs baseline: 2.2719x; 2.2719x over previous
import functools

import jax
import jax.numpy as jnp
from jax import lax
from jax.experimental import pallas as pl
from jax.experimental.pallas import tpu as pltpu
from jax.experimental.pallas import tpu_sc as plsc

N_NODES = 10000
N_EDGES = 320000
D = 128
L = 16
NC = 2
NS = 16
NW = NC * NS
SUB = 128
CHUNK = 256
NSUB = CHUNK // SUB
NCHUNK = N_EDGES // CHUNK
TRIPS = (NCHUNK + NW - 1) // NW
PAIRS = TRIPS // 2
ROWS_LO = 624
ROWS_HI = N_NODES - 15 * ROWS_LO
ZROWS = 16


def _sc_partials(x, packed, pw):
    mesh = plsc.VectorSubcoreMesh(
        core_axis_name="c", subcore_axis_name="s", num_cores=NC, num_subcores=NS
    )

    @functools.partial(
        pl.kernel,
        out_type=jax.ShapeDtypeStruct((NC, N_NODES, D), jnp.float32),
        mesh=mesh,
        scratch_types=[
            pltpu.VMEM_SHARED((N_NODES, D), jnp.float32),
            pltpu.VMEM((2, 4, SUB), jnp.int32),
            pltpu.VMEM((2, 2, SUB), jnp.float32),
            pltpu.VMEM((CHUNK, D), jnp.float32),
            pltpu.VMEM((ZROWS, D), jnp.float32),
            pltpu.SemaphoreType.DMA,
            pltpu.SemaphoreType.DMA,
            pltpu.SemaphoreType.DMA,
            pltpu.SemaphoreType.DMA,
        ],
    )
    def k(x_hbm, pk_hbm, pw_hbm, parts_hbm, acc, pbuf, pwbuf, rows, zbuf,
          sem_g, sem_i0, sem_i1, sem_sc):
        cid = lax.axis_index("c")
        sid = lax.axis_index("s")
        wid = sid * NC + cid
        base_row = sid * ROWS_LO

        zeros = jnp.zeros((L,), jnp.float32)
        for r in range(ZROWS):
            for j in range(D // L):
                zbuf[r, pl.ds(j * L, L)] = zeros

        def zcopy(kk, _):
            pltpu.sync_copy(zbuf, acc.at[pl.ds(base_row + kk * ZROWS, ZROWS)])
            return 0

        n_owned = jnp.where(sid == NS - 1, ROWS_HI, ROWS_LO)
        lax.fori_loop(0, n_owned // ZROWS, zcopy, 0)
        plsc.subcore_barrier()

        my_trips = jnp.where(wid < NCHUNK - (TRIPS - 1) * NW, TRIPS, TRIPS - 1)
        sems = (sem_i0, sem_i1)

        def issue_idx(trip, pb):
            c = trip * NW + wid
            pltpu.async_copy(pk_hbm.at[c], pbuf.at[pb], sems[pb])
            pltpu.async_copy(pw_hbm.at[c], pwbuf.at[pb], sems[pb])

        def wait_idx(pb):
            pltpu.make_async_copy(pk_hbm.at[0], pbuf.at[pb], sems[pb]).wait()
            pltpu.make_async_copy(pw_hbm.at[0], pwbuf.at[pb], sems[pb]).wait()

        def do_trip(trip, pb):
            wait_idx(pb)
            g0 = pltpu.async_copy(x_hbm.at[pbuf.at[pb, 0]],
                                  rows.at[pl.ds(0, SUB)], sem_g)
            g1 = pltpu.async_copy(x_hbm.at[pbuf.at[pb, 1]],
                                  rows.at[pl.ds(SUB, SUB)], sem_g)
            g0.wait()
            g1.wait()

            for j in range(NSUB):
                def scale(g, _):
                    wg = pwbuf[pb, j, pl.ds(g * L, L)]
                    for ee in range(L):
                        e = j * SUB + g * L + ee
                        ws = wg[ee]
                        for q in range(D // L):
                            sl = pl.ds(q * L, L)
                            rows[e, sl] = rows[e, sl] * ws
                    return 0

                lax.fori_loop(0, SUB // L, scale, 0)

            s0 = pltpu.async_copy(rows.at[pl.ds(0, SUB)],
                                  acc.at[pbuf.at[pb, 2]], sem_sc, add=True)
            s1 = pltpu.async_copy(rows.at[pl.ds(SUB, SUB)],
                                  acc.at[pbuf.at[pb, 3]], sem_sc, add=True)
            s0.wait()
            s1.wait()

            @pl.when(trip + 2 < my_trips)
            def _():
                issue_idx(trip + 2, pb)

        issue_idx(0, 0)

        @pl.when(1 < my_trips)
        def _():
            issue_idx(1, 1)

        def pair_body(i2, _):
            t = 2 * i2

            @pl.when(t < my_trips)
            def _():
                do_trip(t, 0)

            @pl.when(t + 1 < my_trips)
            def _():
                do_trip(t + 1, 1)

            return 0

        lax.fori_loop(0, PAIRS, pair_body, 0)
        plsc.subcore_barrier()

        @pl.when(sid < NS - 1)
        def _():
            pltpu.sync_copy(
                acc.at[pl.ds(base_row, ROWS_LO)],
                parts_hbm.at[cid, pl.ds(base_row, ROWS_LO)],
            )

        @pl.when(sid == NS - 1)
        def _():
            pltpu.sync_copy(
                acc.at[pl.ds(15 * ROWS_LO, ROWS_HI)],
                parts_hbm.at[cid, pl.ds(15 * ROWS_LO, ROWS_HI)],
            )

    return k(x, packed, pw)


def _combine(p0, p1):
    def body(a_ref, b_ref, o_ref):
        o_ref[...] = jnp.clip(a_ref[...] + b_ref[...], -20.0, 20.0)

    blk = 1000
    spec = pl.BlockSpec((blk, D), lambda i: (i, 0))
    return pl.pallas_call(
        body,
        grid=(N_NODES // blk,),
        in_specs=[spec, spec],
        out_specs=spec,
        out_shape=jax.ShapeDtypeStruct((N_NODES, D), jnp.float32),
    )(p0, p1)


def kernel(t, x, edge_index, edge_weight):
    src = edge_index[1].reshape(NCHUNK, NSUB, SUB)
    dst = edge_index[0].reshape(NCHUNK, NSUB, SUB)
    pw = edge_weight.reshape(NCHUNK, NSUB, SUB)
    packed = jnp.concatenate([src, dst], axis=1)
    parts = _sc_partials(x, packed, pw)
    return _combine(parts[0], parts[1])


# speedup vs baseline: 2.4207x; 1.0655x over previous
import functools

import jax
import jax.numpy as jnp
from jax import lax
from jax.experimental import pallas as pl
from jax.experimental.pallas import tpu as pltpu
from jax.experimental.pallas import tpu_sc as plsc

N_NODES = 10000
N_EDGES = 320000
D = 128
L = 16
NC = 2
NS = 16
NW = NC * NS
SUB = 128
CHUNK = 256
NSUB = CHUNK // SUB
NCHUNK = N_EDGES // CHUNK
TRIPS = (NCHUNK + NW - 1) // NW
PAIRS = TRIPS // 2
ROWS_LO = 624
ROWS_HI = N_NODES - 15 * ROWS_LO
ZROWS = 16


def _sc_partials(x, packed, pw):
    mesh = plsc.VectorSubcoreMesh(
        core_axis_name="c", subcore_axis_name="s", num_cores=NC, num_subcores=NS
    )

    @functools.partial(
        pl.kernel,
        out_type=jax.ShapeDtypeStruct((NC, N_NODES, D), jnp.float32),
        mesh=mesh,
        scratch_types=[
            pltpu.VMEM_SHARED((N_NODES, D), jnp.float32),
            pltpu.VMEM((2, 2 * NSUB, SUB), jnp.int32),
            pltpu.VMEM((2, NSUB, SUB), jnp.float32),
            pltpu.VMEM((CHUNK, D), jnp.float32),
            pltpu.VMEM((ZROWS, D), jnp.float32),
            pltpu.SemaphoreType.DMA,
            pltpu.SemaphoreType.DMA,
            pltpu.SemaphoreType.DMA,
            pltpu.SemaphoreType.DMA,
        ],
    )
    def k(x_hbm, pk_hbm, pw_hbm, parts_hbm, acc, pbuf, pwbuf, rows, zbuf,
          sem_g, sem_i0, sem_i1, sem_sc):
        cid = lax.axis_index("c")
        sid = lax.axis_index("s")
        wid = sid * NC + cid
        base_row = sid * ROWS_LO

        zeros = jnp.zeros((L,), jnp.float32)
        for r in range(ZROWS):
            for j in range(D // L):
                zbuf[r, pl.ds(j * L, L)] = zeros

        def zcopy(kk, _):
            pltpu.sync_copy(zbuf, acc.at[pl.ds(base_row + kk * ZROWS, ZROWS)])
            return 0

        n_owned = jnp.where(sid == NS - 1, ROWS_HI, ROWS_LO)
        lax.fori_loop(0, n_owned // ZROWS, zcopy, 0)
        plsc.subcore_barrier()

        my_trips = jnp.where(wid < NCHUNK - (TRIPS - 1) * NW, TRIPS, TRIPS - 1)
        sems = (sem_i0, sem_i1)

        def issue_idx(trip, pb):
            c = trip * NW + wid
            pltpu.async_copy(pk_hbm.at[c], pbuf.at[pb], sems[pb])
            pltpu.async_copy(pw_hbm.at[c], pwbuf.at[pb], sems[pb])

        def wait_idx(pb):
            pltpu.make_async_copy(pk_hbm.at[0], pbuf.at[pb], sems[pb]).wait()
            pltpu.make_async_copy(pw_hbm.at[0], pwbuf.at[pb], sems[pb]).wait()

        def do_trip(trip, pb):
            wait_idx(pb)
            gs = [
                pltpu.async_copy(x_hbm.at[pbuf.at[pb, j]],
                                 rows.at[pl.ds(j * SUB, SUB)], sem_g)
                for j in range(NSUB)
            ]
            for g in gs:
                g.wait()

            scs = []
            for j in range(NSUB):
                def scale(g, _):
                    wg = pwbuf[pb, j, pl.ds(g * L, L)]
                    for ee in range(L):
                        e = j * SUB + g * L + ee
                        ws = wg[ee]
                        for q in range(D // L):
                            sl = pl.ds(q * L, L)
                            rows[e, sl] = rows[e, sl] * ws
                    return 0

                lax.fori_loop(0, SUB // L, scale, 0)
                scs.append(
                    pltpu.async_copy(rows.at[pl.ds(j * SUB, SUB)],
                                     acc.at[pbuf.at[pb, NSUB + j]], sem_sc,
                                     add=True)
                )

            @pl.when(trip + 2 < my_trips)
            def _():
                issue_idx(trip + 2, pb)

            for sdesc in scs:
                sdesc.wait()

        issue_idx(0, 0)

        @pl.when(1 < my_trips)
        def _():
            issue_idx(1, 1)

        def pair_body(i2, _):
            t = 2 * i2

            @pl.when(t < my_trips)
            def _():
                do_trip(t, 0)

            @pl.when(t + 1 < my_trips)
            def _():
                do_trip(t + 1, 1)

            return 0

        lax.fori_loop(0, PAIRS, pair_body, 0)
        plsc.subcore_barrier()

        @pl.when(sid < NS - 1)
        def _():
            pltpu.sync_copy(
                acc.at[pl.ds(base_row, ROWS_LO)],
                parts_hbm.at[cid, pl.ds(base_row, ROWS_LO)],
            )

        @pl.when(sid == NS - 1)
        def _():
            pltpu.sync_copy(
                acc.at[pl.ds(15 * ROWS_LO, ROWS_HI)],
                parts_hbm.at[cid, pl.ds(15 * ROWS_LO, ROWS_HI)],
            )

    return k(x, packed, pw)


def _combine(p0, p1):
    def body(a_ref, b_ref, o_ref):
        o_ref[...] = jnp.clip(a_ref[...] + b_ref[...], -20.0, 20.0)

    blk = 1000
    spec = pl.BlockSpec((blk, D), lambda i: (i, 0))
    return pl.pallas_call(
        body,
        grid=(N_NODES // blk,),
        in_specs=[spec, spec],
        out_specs=spec,
        out_shape=jax.ShapeDtypeStruct((N_NODES, D), jnp.float32),
    )(p0, p1)


def kernel(t, x, edge_index, edge_weight):
    src = edge_index[1].reshape(NCHUNK, NSUB, SUB)
    dst = edge_index[0].reshape(NCHUNK, NSUB, SUB)
    pw = edge_weight.reshape(NCHUNK, NSUB, SUB)
    packed = jnp.concatenate([src, dst], axis=1)
    parts = _sc_partials(x, packed, pw)
    return _combine(parts[0], parts[1])
